# baseline (speedup 1.0000x reference)
import jax
import jax.numpy as jnp
from jax import lax
from jax.experimental import pallas as pl
from jax.experimental.pallas import tpu as pltpu
from jax.experimental.pallas import tpu_sc as plsc

D = 128
E = 320000
W = D // 2

NC = 2
NS = 16
L = 16
NW = NC * NS
EPW = E // NW
K = 80
NCHUNK = EPW // K
NGROUP = K // L

WUNROLL = 16


def _dot_chunk(u_buf, i_buf, out_v, out_base):
    iota = lax.iota(jnp.int32, L)
    rots = [jnp.bitwise_and(iota + k, L - 1) for k in range(WUNROLL)]

    def group_body(g, _):
        e_ids = g * L + iota

        zero = jnp.zeros((L,), jnp.float32)
        accs = [zero] * 4
        q = 0
        for t in range(W // WUNROLL):
            wv = jnp.broadcast_to(t * WUNROLL, (L,)).astype(jnp.int32)
            for k4 in range(WUNROLL // 4):
                ps = []
                for k in range(4 * k4, 4 * k4 + 4):
                    wvk = wv + rots[k]
                    u = plsc.load_gather(u_buf, [e_ids, wvk])
                    v = plsc.load_gather(i_buf, [e_ids, wvk])
                    ps.append(plsc.bitcast(u, jnp.bfloat16) *
                              plsc.bitcast(v, jnp.bfloat16))
                p = (ps[0] + ps[1]) + (ps[2] + ps[3])
                p_lo, p_hi = plsc.unpack(p, format=plsc.PackFormat.INTERLEAVED)
                accs[q % 2] = accs[q % 2] + p_lo
                accs[2 + q % 2] = accs[2 + q % 2] + p_hi
                q += 1
        out_v[pl.ds(out_base + g * L, L)] = (
            (accs[0] + accs[1]) + (accs[2] + accs[3]))
        return 0

    lax.fori_loop(0, NGROUP, group_body, 0)


def _sc_body(user_hbm, item_hbm, edge_hbm, out_hbm,
             idx_u, idx_i, out_v,
             u_a, i_a, u_b, i_b,
             sem_ua, sem_ia, sem_ub, sem_ib):
    wid = lax.axis_index("s") * NC + lax.axis_index("c")
    base = wid * EPW

    pltpu.sync_copy(edge_hbm.at[0, pl.ds(base, EPW)], idx_u)
    pltpu.sync_copy(edge_hbm.at[1, pl.ds(base, EPW)], idx_i)

    def fire(c, u_dst, i_dst, s_u, s_i):
        off = c * K
        pltpu.async_copy(user_hbm.at[idx_u.at[pl.ds(off, K)]], u_dst, s_u)
        pltpu.async_copy(item_hbm.at[idx_i.at[pl.ds(off, K)]], i_dst, s_i)

    def drain(u_dst, i_dst, s_u, s_i):
        pltpu.make_async_copy(user_hbm.at[idx_u.at[pl.ds(0, K)]], u_dst, s_u).wait()
        pltpu.make_async_copy(item_hbm.at[idx_i.at[pl.ds(0, K)]], i_dst, s_i).wait()

    fire(0, u_a, i_a, sem_ua, sem_ia)

    def pair_body(cc, _):
        a = cc * 2
        b = a + 1
        fire(b, u_b, i_b, sem_ub, sem_ib)
        drain(u_a, i_a, sem_ua, sem_ia)
        _dot_chunk(u_a, i_a, out_v, a * K)
        fire(a + 2, u_a, i_a, sem_ua, sem_ia)
        drain(u_b, i_b, sem_ub, sem_ib)
        _dot_chunk(u_b, i_b, out_v, b * K)
        return 0

    lax.fori_loop(0, (NCHUNK - 1) // 2, pair_body, 0)
    drain(u_a, i_a, sem_ua, sem_ia)
    _dot_chunk(u_a, i_a, out_v, (NCHUNK - 1) * K)

    pltpu.sync_copy(out_v, out_hbm.at[pl.ds(base, EPW)])


@jax.jit
def _run(user_pack, item_pack, edge_index):
    mesh = plsc.VectorSubcoreMesh(core_axis_name="c", subcore_axis_name="s")
    kern = pl.kernel(
        _sc_body,
        out_type=jax.ShapeDtypeStruct((E,), jnp.float32),
        mesh=mesh,
        compiler_params=pltpu.CompilerParams(needs_layout_passes=False,
                                             use_tc_tiling_on_sc=False),
        scratch_types=[
            pltpu.VMEM((EPW,), jnp.int32),
            pltpu.VMEM((EPW,), jnp.int32),
            pltpu.VMEM((EPW,), jnp.float32),
            pltpu.VMEM((K, W), jnp.int32),
            pltpu.VMEM((K, W), jnp.int32),
            pltpu.VMEM((K, W), jnp.int32),
            pltpu.VMEM((K, W), jnp.int32),
            pltpu.SemaphoreType.DMA,
            pltpu.SemaphoreType.DMA,
            pltpu.SemaphoreType.DMA,
            pltpu.SemaphoreType.DMA,
        ],
    )
    return kern(user_pack, item_pack, edge_index)


def _pack_bf16(table):
    x = lax.bitcast_convert_type(table, jnp.uint32)
    r = (x + 0x7FFF + ((x >> 16) & 1)) >> 16
    lo = r[:, :W]
    hi = r[:, W:]
    return lax.bitcast_convert_type(lo | (hi << 16), jnp.int32)


def kernel(user_embedding, item_embedding, edge_index):
    return _run(_pack_bf16(user_embedding), _pack_bf16(item_embedding),
                edge_index.astype(jnp.int32))


# speedup vs baseline: 2.1413x; 2.1413x over previous
import jax
import jax.numpy as jnp
from jax import lax
from jax.experimental import pallas as pl
from jax.experimental.pallas import tpu as pltpu
from jax.experimental.pallas import tpu_sc as plsc

D = 128
E = 320000
W = D // 2

NC = 2
NS = 16
L = 16
NW = NC * NS
EPW = E // NW
K = 80
NCHUNK = EPW // K
NGROUP = K // L

WUNROLL = 16


def _dot_chunk(u_buf, i_buf, out_v, out_base):
    iota = lax.iota(jnp.int32, L)
    rots = [jnp.bitwise_and(iota + k, L - 1) for k in range(WUNROLL)]

    def group_body(g, _):
        e_ids = g * L + iota

        def w_step(t, carry):
            accs = list(carry[:4])
            wv = carry[4]
            for k4 in range(WUNROLL // 4):
                ps = []
                for k in range(4 * k4, 4 * k4 + 4):
                    wvk = wv + rots[k]
                    u = plsc.load_gather(u_buf, [e_ids, wvk])
                    v = plsc.load_gather(i_buf, [e_ids, wvk])
                    ps.append(plsc.bitcast(u, jnp.bfloat16) *
                              plsc.bitcast(v, jnp.bfloat16))
                p = (ps[0] + ps[1]) + (ps[2] + ps[3])
                p_lo, p_hi = plsc.unpack(p, format=plsc.PackFormat.INTERLEAVED)
                accs[k4 % 2] = accs[k4 % 2] + p_lo
                accs[2 + k4 % 2] = accs[2 + k4 % 2] + p_hi
            return (*accs, wv + WUNROLL)

        zero = jnp.zeros((L,), jnp.float32)
        wv0 = jnp.zeros((L,), jnp.int32)
        carry = lax.fori_loop(0, W // WUNROLL, w_step, (zero,) * 4 + (wv0,))
        accs = carry[:4]
        out_v[pl.ds(out_base + g * L, L)] = (
            (accs[0] + accs[1]) + (accs[2] + accs[3]))
        return 0

    lax.fori_loop(0, NGROUP, group_body, 0)


def _sc_body(user_hbm, item_hbm, edge_hbm, out_hbm,
             idx_u, idx_i, out_v,
             u_a, i_a, u_b, i_b,
             sem_ua, sem_ia, sem_ub, sem_ib):
    wid = lax.axis_index("s") * NC + lax.axis_index("c")
    base = wid * EPW

    pltpu.sync_copy(edge_hbm.at[0, pl.ds(base, EPW)], idx_u)
    pltpu.sync_copy(edge_hbm.at[1, pl.ds(base, EPW)], idx_i)

    def fire(c, u_dst, i_dst, s_u, s_i):
        off = c * K
        pltpu.async_copy(user_hbm.at[idx_u.at[pl.ds(off, K)]], u_dst, s_u)
        pltpu.async_copy(item_hbm.at[idx_i.at[pl.ds(off, K)]], i_dst, s_i)

    def drain(u_dst, i_dst, s_u, s_i):
        pltpu.make_async_copy(user_hbm.at[idx_u.at[pl.ds(0, K)]], u_dst, s_u).wait()
        pltpu.make_async_copy(item_hbm.at[idx_i.at[pl.ds(0, K)]], i_dst, s_i).wait()

    fire(0, u_a, i_a, sem_ua, sem_ia)

    def pair_body(cc, _):
        a = cc * 2
        b = a + 1
        fire(b, u_b, i_b, sem_ub, sem_ib)
        drain(u_a, i_a, sem_ua, sem_ia)
        _dot_chunk(u_a, i_a, out_v, a * K)
        fire(a + 2, u_a, i_a, sem_ua, sem_ia)
        drain(u_b, i_b, sem_ub, sem_ib)
        _dot_chunk(u_b, i_b, out_v, b * K)
        return 0

    lax.fori_loop(0, (NCHUNK - 1) // 2, pair_body, 0)
    drain(u_a, i_a, sem_ua, sem_ia)
    _dot_chunk(u_a, i_a, out_v, (NCHUNK - 1) * K)

    pltpu.sync_copy(out_v, out_hbm.at[pl.ds(base, EPW)])


@jax.jit
def _run(user_pack, item_pack, edge_index):
    mesh = plsc.VectorSubcoreMesh(core_axis_name="c", subcore_axis_name="s")
    kern = pl.kernel(
        _sc_body,
        out_type=jax.ShapeDtypeStruct((E,), jnp.float32),
        mesh=mesh,
        compiler_params=pltpu.CompilerParams(needs_layout_passes=False,
                                             use_tc_tiling_on_sc=False),
        scratch_types=[
            pltpu.VMEM((EPW,), jnp.int32),
            pltpu.VMEM((EPW,), jnp.int32),
            pltpu.VMEM((EPW,), jnp.float32),
            pltpu.VMEM((K, W), jnp.int32),
            pltpu.VMEM((K, W), jnp.int32),
            pltpu.VMEM((K, W), jnp.int32),
            pltpu.VMEM((K, W), jnp.int32),
            pltpu.SemaphoreType.DMA,
            pltpu.SemaphoreType.DMA,
            pltpu.SemaphoreType.DMA,
            pltpu.SemaphoreType.DMA,
        ],
    )
    return kern(user_pack, item_pack, edge_index)


def _pack_bf16(table):
    x = lax.bitcast_convert_type(table, jnp.uint32)
    r = (x + 0x7FFF + ((x >> 16) & 1)) >> 16
    lo = r[:, :W]
    hi = r[:, W:]
    return lax.bitcast_convert_type(lo | (hi << 16), jnp.int32)


def kernel(user_embedding, item_embedding, edge_index):
    return _run(_pack_bf16(user_embedding), _pack_bf16(item_embedding),
                edge_index.astype(jnp.int32))


# speedup vs baseline: 2.2562x; 1.0537x over previous
import jax
import jax.numpy as jnp
from jax import lax
from jax.experimental import pallas as pl
from jax.experimental.pallas import tpu as pltpu
from jax.experimental.pallas import tpu_sc as plsc

D = 128
E = 320000
W = D // 2

NC = 2
NS = 16
L = 16
NW = NC * NS
EPW = E // NW
K = 80
NCHUNK = EPW // K
NGROUP = K // L

WUNROLL = 32


def _dot_chunk(u_buf, i_buf, out_v, out_base):
    iota = lax.iota(jnp.int32, L)
    rots = [jnp.bitwise_and(iota + k, L - 1) for k in range(L)]
    rots = rots + rots

    def group_body(g, _):
        e_ids = g * L + iota

        def w_step(t, carry):
            accs = list(carry[:4])
            wv = carry[4]
            for k2 in range(WUNROLL // 2):
                ps = []
                for k in (2 * k2, 2 * k2 + 1):
                    wvk = wv + rots[k]
                    u = plsc.load_gather(u_buf, [e_ids, wvk])
                    v = plsc.load_gather(i_buf, [e_ids, wvk])
                    ps.append(plsc.bitcast(u, jnp.bfloat16) *
                              plsc.bitcast(v, jnp.bfloat16))
                p = ps[0] + ps[1]
                p_lo, p_hi = plsc.unpack(p, format=plsc.PackFormat.INTERLEAVED)
                accs[k2 % 2] = accs[k2 % 2] + p_lo
                accs[2 + k2 % 2] = accs[2 + k2 % 2] + p_hi
            return (*accs, wv + WUNROLL)

        zero = jnp.zeros((L,), jnp.float32)
        wv0 = jnp.zeros((L,), jnp.int32)
        carry = lax.fori_loop(0, W // WUNROLL, w_step, (zero,) * 4 + (wv0,))
        accs = carry[:4]
        out_v[pl.ds(out_base + g * L, L)] = (
            (accs[0] + accs[1]) + (accs[2] + accs[3]))
        return 0

    lax.fori_loop(0, NGROUP, group_body, 0)


def _sc_body(user_hbm, item_hbm, edge_hbm, out_hbm,
             idx_u, idx_i, out_v,
             u_a, i_a, u_b, i_b,
             sem_ua, sem_ia, sem_ub, sem_ib):
    wid = lax.axis_index("s") * NC + lax.axis_index("c")
    base = wid * EPW

    pltpu.sync_copy(edge_hbm.at[0, pl.ds(base, EPW)], idx_u)
    pltpu.sync_copy(edge_hbm.at[1, pl.ds(base, EPW)], idx_i)

    def fire(c, u_dst, i_dst, s_u, s_i):
        off = c * K
        pltpu.async_copy(user_hbm.at[idx_u.at[pl.ds(off, K)]], u_dst, s_u)
        pltpu.async_copy(item_hbm.at[idx_i.at[pl.ds(off, K)]], i_dst, s_i)

    def drain(u_dst, i_dst, s_u, s_i):
        pltpu.make_async_copy(user_hbm.at[idx_u.at[pl.ds(0, K)]], u_dst, s_u).wait()
        pltpu.make_async_copy(item_hbm.at[idx_i.at[pl.ds(0, K)]], i_dst, s_i).wait()

    fire(0, u_a, i_a, sem_ua, sem_ia)

    def pair_body(cc, _):
        a = cc * 2
        b = a + 1
        fire(b, u_b, i_b, sem_ub, sem_ib)
        drain(u_a, i_a, sem_ua, sem_ia)
        _dot_chunk(u_a, i_a, out_v, a * K)
        fire(a + 2, u_a, i_a, sem_ua, sem_ia)
        drain(u_b, i_b, sem_ub, sem_ib)
        _dot_chunk(u_b, i_b, out_v, b * K)
        return 0

    lax.fori_loop(0, (NCHUNK - 1) // 2, pair_body, 0)
    drain(u_a, i_a, sem_ua, sem_ia)
    _dot_chunk(u_a, i_a, out_v, (NCHUNK - 1) * K)

    pltpu.sync_copy(out_v, out_hbm.at[pl.ds(base, EPW)])


@jax.jit
def _run(user_pack, item_pack, edge_index):
    mesh = plsc.VectorSubcoreMesh(core_axis_name="c", subcore_axis_name="s")
    kern = pl.kernel(
        _sc_body,
        out_type=jax.ShapeDtypeStruct((E,), jnp.float32),
        mesh=mesh,
        compiler_params=pltpu.CompilerParams(needs_layout_passes=False,
                                             use_tc_tiling_on_sc=False),
        scratch_types=[
            pltpu.VMEM((EPW,), jnp.int32),
            pltpu.VMEM((EPW,), jnp.int32),
            pltpu.VMEM((EPW,), jnp.float32),
            pltpu.VMEM((K, W), jnp.int32),
            pltpu.VMEM((K, W), jnp.int32),
            pltpu.VMEM((K, W), jnp.int32),
            pltpu.VMEM((K, W), jnp.int32),
            pltpu.SemaphoreType.DMA,
            pltpu.SemaphoreType.DMA,
            pltpu.SemaphoreType.DMA,
            pltpu.SemaphoreType.DMA,
        ],
    )
    return kern(user_pack, item_pack, edge_index)


def _pack_bf16(table):
    x = lax.bitcast_convert_type(table, jnp.uint32)
    r = (x + 0x7FFF + ((x >> 16) & 1)) >> 16
    lo = r[:, :W]
    hi = r[:, W:]
    return lax.bitcast_convert_type(lo | (hi << 16), jnp.int32)


def kernel(user_embedding, item_embedding, edge_index):
    return _run(_pack_bf16(user_embedding), _pack_bf16(item_embedding),
                edge_index.astype(jnp.int32))
